# baseline (speedup 1.0000x reference)
import jax
import jax.numpy as jnp
from jax.experimental import pallas as pl
from jax.experimental.pallas import tpu as pltpu

_BLOCK_N = 5000


def _gnmax_body(x_ref, w_ref, o_ref):
    i = pl.program_id(0)
    h = jnp.dot(x_ref[...], w_ref[...], preferred_element_type=jnp.float32)
    m = jnp.max(h)

    @pl.when(i == 0)
    def _init():
        o_ref[0] = m

    @pl.when(i > 0)
    def _acc():
        o_ref[0] = jnp.maximum(o_ref[0], m)


def kernel(x, W, b):
    n, d = x.shape
    x2 = x.reshape(n // 2, 2 * d)
    w = W.reshape(d)
    w2 = jnp.zeros((2 * d, 2), jnp.float32)
    w2 = w2.at[:d, 0].set(w).at[d:, 1].set(w)
    grid = (n // 2) // _BLOCK_N
    m = pl.pallas_call(
        _gnmax_body,
        grid=(grid,),
        in_specs=[
            pl.BlockSpec((_BLOCK_N, 2 * d), lambda i: (i, 0)),
            pl.BlockSpec((2 * d, 2), lambda i: (0, 0)),
        ],
        out_specs=pl.BlockSpec(memory_space=pltpu.SMEM),
        out_shape=jax.ShapeDtypeStruct((1,), jnp.float32),
    )(x2, w2)
    return m + b


# speedup vs baseline: 6.3192x; 6.3192x over previous
import jax
import jax.numpy as jnp
from jax.experimental import pallas as pl
from jax.experimental.pallas import tpu as pltpu

_CW = 12800
_NBUF = 4


def _make_body(d, n_aligned, tail):
    widths = [_CW] * (n_aligned // _CW)
    if n_aligned % _CW:
        widths.append(n_aligned % _CW)
    starts = [sum(widths[:i]) for i in range(len(widths))]
    chunks = list(zip(starts, widths))
    nch = len(chunks)
    nbuf = min(_NBUF, nch)

    def body(xt_hbm, w_ref, tail_ref, o_ref, *scr):
        bufs, sems = scr[:nbuf], scr[nbuf:]

        def dma(c, s):
            st, wd = chunks[c]
            return pltpu.make_async_copy(
                xt_hbm.at[:, pl.ds(st, wd)],
                bufs[s].at[:, pl.ds(0, wd)],
                sems[s],
            )

        for s in range(nbuf):
            dma(s, s).start()
        for c in range(nch):
            s = c % nbuf
            dma(c, s).wait()
            wd = chunks[c][1]
            y = bufs[s][:, :wd] * w_ref[...]
            m = jnp.max(jnp.sum(y, axis=0))
            if c == 0:
                if tail:
                    mt = jnp.max(
                        jnp.dot(tail_ref[...], w_ref[...],
                                preferred_element_type=jnp.float32))
                    m = jnp.maximum(m, mt)
                o_ref[0] = m
            else:
                o_ref[0] = jnp.maximum(o_ref[0], m)
            nxt = c + nbuf
            if nxt < nch:
                dma(nxt, s).start()

    return body, nbuf


def kernel(x, W, b):
    n, d = x.shape
    n_aligned = (n // 128) * 128
    tail = n - n_aligned
    xt = x.T
    wcol = W.T
    x_tail = jax.lax.slice(x, (n_aligned, 0), (n, d)) if tail else x[:8]
    body, nbuf = _make_body(d, n_aligned, tail)
    m = pl.pallas_call(
        body,
        in_specs=[
            pl.BlockSpec(memory_space=pl.ANY),
            pl.BlockSpec(memory_space=pltpu.MemorySpace.VMEM),
            pl.BlockSpec(memory_space=pltpu.MemorySpace.VMEM),
        ],
        out_specs=pl.BlockSpec(memory_space=pltpu.SMEM),
        out_shape=jax.ShapeDtypeStruct((1,), jnp.float32),
        scratch_shapes=[pltpu.VMEM((d, _CW), jnp.float32)] * nbuf
        + [pltpu.SemaphoreType.DMA] * nbuf,
    )(xt, wcol, x_tail)
    return m + b


# speedup vs baseline: 6.7108x; 1.0620x over previous
import jax
import jax.numpy as jnp
from jax.experimental import pallas as pl
from jax.experimental.pallas import tpu as pltpu

_NBUF = 4


def _make_body(d, n, nbuf):
    nslab = d // 8

    def body(xt_hbm, w_ref, o_ref, acc, *scr):
        bufs, sems = scr[:nbuf], scr[nbuf:]

        def dma(c, s):
            return pltpu.make_async_copy(
                xt_hbm.at[pl.ds(c * 8, 8), :], bufs[s], sems[s])

        for s in range(nbuf):
            dma(s, s).start()
        for c in range(nslab):
            s = c % nbuf
            dma(c, s).wait()
            y = bufs[s][...] * w_ref[pl.ds(c * 8, 8), :]
            if c == 0:
                acc[...] = y
            elif c == nslab - 1:
                o_ref[0] = jnp.max(jnp.sum(acc[...] + y, axis=0))
            else:
                acc[...] += y
            nxt = c + nbuf
            if nxt < nslab:
                dma(nxt, s).start()

    return body


def kernel(x, W, b):
    n, d = x.shape
    xt = x.T
    wcol = W.T
    nbuf = min(_NBUF, d // 8)
    m = pl.pallas_call(
        _make_body(d, n, nbuf),
        in_specs=[
            pl.BlockSpec(memory_space=pl.ANY),
            pl.BlockSpec(memory_space=pltpu.MemorySpace.VMEM),
        ],
        out_specs=pl.BlockSpec(memory_space=pltpu.SMEM),
        out_shape=jax.ShapeDtypeStruct((1,), jnp.float32),
        scratch_shapes=[pltpu.VMEM((8, n), jnp.float32)]
        + [pltpu.VMEM((8, n), jnp.float32)] * nbuf
        + [pltpu.SemaphoreType.DMA] * nbuf,
    )(xt, wcol)
    return m + b
